# baseline (speedup 1.0000x reference)
import functools

import jax
import jax.numpy as jnp
from jax import lax
from jax.experimental import pallas as pl
from jax.experimental.pallas import tpu as pltpu
from jax.experimental.pallas import tpu_sc as plsc

B = 16384
NC, NS = 2, 16
NW = NC * NS
S = 2
SLICE = B // S
BPW = SLICE // NW
CHUNK = 64
NCH = BPW // CHUNK


def _sc_gather(s, item, gidx, tidx, item_table, genre_table, tags_table):
    mesh = plsc.VectorSubcoreMesh(core_axis_name="c", subcore_axis_name="s")

    @functools.partial(
        pl.kernel,
        mesh=mesh,
        out_type=(
            jax.ShapeDtypeStruct((SLICE, 128), jnp.float32),
            jax.ShapeDtypeStruct((SLICE, 256), jnp.float32),
            jax.ShapeDtypeStruct((SLICE, 256), jnp.float32),
        ),
        scratch_types=(
            pltpu.VMEM((BPW,), jnp.int32),
            pltpu.VMEM((BPW,), jnp.int32),
            pltpu.VMEM((BPW,), jnp.int32),
            pltpu.VMEM((CHUNK, 128), jnp.float32),
            pltpu.VMEM((CHUNK, 128), jnp.float32),
            pltpu.VMEM((CHUNK, 128), jnp.float32),
            pltpu.VMEM((CHUNK, 256), jnp.float32),
            pltpu.VMEM((CHUNK, 256), jnp.float32),
            pltpu.VMEM((CHUNK, 256), jnp.float32),
            pltpu.VMEM((CHUNK, 256), jnp.float32),
            pltpu.VMEM((CHUNK, 256), jnp.float32),
            pltpu.VMEM((CHUNK, 256), jnp.float32),
        ) + (pltpu.SemaphoreType.DMA,) * 21,
    )
    def k(item_h, gidx_h, tidx_h, itab_h, gtab_h, ttab_h,
          v_out, g_out, t_out,
          ii_v, gi_v, ti_v, ib0, ib1, ib2, gb0, gb1, gb2, tb0, tb1, tb2,
          *sems):
        sems = list(sems)
        wid = lax.axis_index("s") * NC + lax.axis_index("c")
        base = wid * BPW
        src = s * SLICE + base
        a0 = pltpu.async_copy(item_h.at[pl.ds(src, BPW)], ii_v, sems[0])
        a1 = pltpu.async_copy(gidx_h.at[pl.ds(src, BPW)], gi_v, sems[1])
        a2 = pltpu.async_copy(tidx_h.at[pl.ds(src, BPW)], ti_v, sems[2])

        def gath(tab, idx, buf, c, sem):
            return pltpu.async_copy(
                tab.at[idx.at[pl.ds(c * CHUNK, CHUNK)]], buf, sem)

        def put(buf, out, c, sem):
            return pltpu.async_copy(
                buf, out.at[pl.ds(base + c * CHUNK, CHUNK)], sem)

        NBUF = 3
        lanes = (
            (itab_h, ii_v, v_out, (ib0, ib1, ib2), a0, sems[3:9]),
            (gtab_h, gi_v, g_out, (gb0, gb1, gb2), a1, sems[9:15]),
            (ttab_h, ti_v, t_out, (tb0, tb1, tb2), a2, sems[15:21]),
        )
        gets = [[None] * NCH for _ in lanes]
        puts = [[None] * NCH for _ in lanes]
        for li, (tab, idx, out, bufs, a, sm) in enumerate(lanes):
            a.wait()
            for c in range(min(NBUF, NCH)):
                gets[li][c] = gath(tab, idx, bufs[c % NBUF], c, sm[c % NBUF])
        for c in range(NCH):
            for li, (tab, idx, out, bufs, a, sm) in enumerate(lanes):
                gets[li][c].wait()
                puts[li][c] = put(bufs[c % NBUF], out, c,
                                  sm[NBUF + c % NBUF])
                if c + NBUF < NCH:
                    puts[li][c].wait()
                    gets[li][c + NBUF] = gath(tab, idx, bufs[c % NBUF],
                                              c + NBUF, sm[c % NBUF])
        for li in range(len(lanes)):
            for c in range(NCH):
                if c + NBUF >= NCH:
                    puts[li][c].wait()

    return k(item, gidx, tidx, item_table, genre_table, tags_table)


BLK = 4096


def _tc_mlp(s, v, g, t, price2, W1bf, pw1, b1c, W2, b2, W3, b3, W4, b4):
    bf = jnp.bfloat16
    nblk = SLICE // BLK

    def body(v_ref, g_ref, t_ref, p_ref, w1, pw, b1r,
             w2, b2r, w3, b3r, w4, b4r, o_ref):
        dot = lambda a, w: jnp.dot(a, w, preferred_element_type=jnp.float32)
        x = (dot(v_ref[...].astype(bf), w1[0:128])
             + dot(g_ref[...].astype(bf), w1[128:384])
             + dot(t_ref[...].astype(bf), w1[384:640]))
        x += p_ref[...] * pw[...]
        x = jnp.maximum(x + b1r[...], 0.0)
        x = jnp.maximum(dot(x.astype(bf), w2[...]) + b2r[...], 0.0)
        x = jnp.maximum(dot(x.astype(bf), w3[...]) + b3r[...], 0.0)
        o_ref[...] = jnp.maximum(dot(x.astype(bf), w4[...]) + b4r[...], 0.0)

    full = lambda sh: pl.BlockSpec(sh, lambda i: (0, 0))
    return pl.pallas_call(
        body,
        grid=(nblk,),
        in_specs=[
            pl.BlockSpec((BLK, 128), lambda i: (i, 0)),
            pl.BlockSpec((BLK, 256), lambda i: (i, 0)),
            pl.BlockSpec((BLK, 256), lambda i: (i, 0)),
            pl.BlockSpec((BLK, 1), lambda i, s=s: (i + s * nblk, 0)),
            full((768, 384)),
            full((1, 384)), full((1, 384)),
            full((384, 192)), full((1, 192)),
            full((192, 96)), full((1, 96)),
            full((96, 48)), full((1, 48)),
        ],
        out_specs=pl.BlockSpec((BLK, 48), lambda i: (i, 0)),
        out_shape=jax.ShapeDtypeStruct((SLICE, 48), jnp.float32),
        compiler_params=pltpu.CompilerParams(
            dimension_semantics=("arbitrary",)),
    )(v, g, t, price2, W1bf, pw1, b1c, W2, b2, W3, b3, W4, b4)


def kernel(item, genre_indices, genre_offsets, tags_indices, tags_offsets,
           price, item_table, genre_table, tags_table, price_W, price_b,
           W1, b1, W2, b2, W3, b3, W4, b4):
    del genre_offsets, tags_offsets
    item = item.astype(jnp.int32)
    gidx = genre_indices.astype(jnp.int32)
    tidx = tags_indices.astype(jnp.int32)
    bf = jnp.bfloat16
    W1p = W1[640:]
    pw1 = price_W @ W1p
    b1c = (b1 + price_b @ W1p)[None, :]
    W1bf = W1.astype(bf)
    W2bf, W3bf, W4bf = W2.astype(bf), W3.astype(bf), W4.astype(bf)
    price2 = price[:, None]
    outs = []
    for s in range(S):
        v, g, t = _sc_gather(s, item, gidx, tidx,
                             item_table, genre_table, tags_table)
        outs.append(_tc_mlp(s, v, g, t, price2,
                            W1bf, pw1, b1c, W2bf, b2[None, :],
                            W3bf, b3[None, :], W4bf, b4[None, :]))
    return jnp.concatenate(outs, axis=0)


# speedup vs baseline: 1.0018x; 1.0018x over previous
import functools

import jax
import jax.numpy as jnp
from jax import lax
from jax.experimental import pallas as pl
from jax.experimental.pallas import tpu as pltpu
from jax.experimental.pallas import tpu_sc as plsc

B = 16384
NC, NS = 2, 16
NW = NC * NS
S = 2
SLICE = B // S
BPW = SLICE // NW
CHUNK = 64
NCH = BPW // CHUNK


def _sc_gather(s, item, gidx, tidx, item_table, genre_table, tags_table):
    mesh = plsc.VectorSubcoreMesh(core_axis_name="c", subcore_axis_name="s")

    @functools.partial(
        pl.kernel,
        mesh=mesh,
        out_type=(
            jax.ShapeDtypeStruct((SLICE, 128), jnp.float32),
            jax.ShapeDtypeStruct((SLICE, 256), jnp.float32),
            jax.ShapeDtypeStruct((SLICE, 256), jnp.float32),
        ),
        scratch_types=(
            pltpu.VMEM((BPW,), jnp.int32),
            pltpu.VMEM((BPW,), jnp.int32),
            pltpu.VMEM((BPW,), jnp.int32),
            pltpu.VMEM((CHUNK, 128), jnp.float32),
            pltpu.VMEM((CHUNK, 128), jnp.float32),
            pltpu.VMEM((CHUNK, 128), jnp.float32),
            pltpu.VMEM((CHUNK, 256), jnp.float32),
            pltpu.VMEM((CHUNK, 256), jnp.float32),
            pltpu.VMEM((CHUNK, 256), jnp.float32),
            pltpu.VMEM((CHUNK, 256), jnp.float32),
            pltpu.VMEM((CHUNK, 256), jnp.float32),
            pltpu.VMEM((CHUNK, 256), jnp.float32),
        ) + (pltpu.SemaphoreType.DMA,) * 27,
    )
    def k(item_h, gidx_h, tidx_h, itab_h, gtab_h, ttab_h,
          v_out, g_out, t_out,
          ii_v, gi_v, ti_v, ib0, ib1, ib2, gb0, gb1, gb2, tb0, tb1, tb2,
          *sems):
        sems = list(sems)
        wid = lax.axis_index("s") * NC + lax.axis_index("c")
        base = wid * BPW
        src = s * SLICE + base
        a0 = pltpu.async_copy(item_h.at[pl.ds(src, BPW)], ii_v, sems[0])
        a1 = pltpu.async_copy(gidx_h.at[pl.ds(src, BPW)], gi_v, sems[1])
        a2 = pltpu.async_copy(tidx_h.at[pl.ds(src, BPW)], ti_v, sems[2])

        def gath(tab, idx, buf, c, sem):
            return pltpu.async_copy(
                tab.at[idx.at[pl.ds(c * CHUNK, CHUNK)]], buf, sem)

        def put(buf, out, c, sem):
            return pltpu.async_copy(
                buf, out.at[pl.ds(base + c * CHUNK, CHUNK)], sem)

        NBUF = 3
        lanes = (
            (itab_h, ii_v, v_out, (ib0, ib1, ib2), a0, sems[3:11]),
            (gtab_h, gi_v, g_out, (gb0, gb1, gb2), a1, sems[11:19]),
            (ttab_h, ti_v, t_out, (tb0, tb1, tb2), a2, sems[19:27]),
        )
        gets = [[None] * NCH for _ in lanes]
        puts = [[None] * NCH for _ in lanes]
        for li, (tab, idx, out, bufs, a, sm) in enumerate(lanes):
            a.wait()
            for c in range(min(NBUF, NCH)):
                gets[li][c] = gath(tab, idx, bufs[c % NBUF], c, sm[c])
        for c in range(NCH):
            for li, (tab, idx, out, bufs, a, sm) in enumerate(lanes):
                gets[li][c].wait()
                puts[li][c] = put(bufs[c % NBUF], out, c, sm[NCH + c])
                if c + NBUF < NCH:
                    puts[li][c].wait()
                    gets[li][c + NBUF] = gath(tab, idx, bufs[c % NBUF],
                                              c + NBUF, sm[c + NBUF])
        for li in range(len(lanes)):
            for c in range(NCH):
                if c + NBUF >= NCH:
                    puts[li][c].wait()

    return k(item, gidx, tidx, item_table, genre_table, tags_table)


BLK = 4096


def _tc_mlp(s, v, g, t, price2, W1bf, pw1, b1c, W2, b2, W3, b3, W4, b4):
    bf = jnp.bfloat16
    nblk = SLICE // BLK

    def body(v_ref, g_ref, t_ref, p_ref, w1, pw, b1r,
             w2, b2r, w3, b3r, w4, b4r, o_ref):
        dot = lambda a, w: jnp.dot(a, w, preferred_element_type=jnp.float32)
        x = (dot(v_ref[...].astype(bf), w1[0:128])
             + dot(g_ref[...].astype(bf), w1[128:384])
             + dot(t_ref[...].astype(bf), w1[384:640]))
        x += p_ref[...] * pw[...]
        x = jnp.maximum(x + b1r[...], 0.0)
        x = jnp.maximum(dot(x.astype(bf), w2[...]) + b2r[...], 0.0)
        x = jnp.maximum(dot(x.astype(bf), w3[...]) + b3r[...], 0.0)
        o_ref[...] = jnp.maximum(dot(x.astype(bf), w4[...]) + b4r[...], 0.0)

    full = lambda sh: pl.BlockSpec(sh, lambda i: (0, 0))
    return pl.pallas_call(
        body,
        grid=(nblk,),
        in_specs=[
            pl.BlockSpec((BLK, 128), lambda i: (i, 0)),
            pl.BlockSpec((BLK, 256), lambda i: (i, 0)),
            pl.BlockSpec((BLK, 256), lambda i: (i, 0)),
            pl.BlockSpec((BLK, 1), lambda i, s=s: (i + s * nblk, 0)),
            full((768, 384)),
            full((1, 384)), full((1, 384)),
            full((384, 192)), full((1, 192)),
            full((192, 96)), full((1, 96)),
            full((96, 48)), full((1, 48)),
        ],
        out_specs=pl.BlockSpec((BLK, 48), lambda i: (i, 0)),
        out_shape=jax.ShapeDtypeStruct((SLICE, 48), jnp.float32),
        compiler_params=pltpu.CompilerParams(
            dimension_semantics=("arbitrary",)),
    )(v, g, t, price2, W1bf, pw1, b1c, W2, b2, W3, b3, W4, b4)


def kernel(item, genre_indices, genre_offsets, tags_indices, tags_offsets,
           price, item_table, genre_table, tags_table, price_W, price_b,
           W1, b1, W2, b2, W3, b3, W4, b4):
    del genre_offsets, tags_offsets
    item = item.astype(jnp.int32)
    gidx = genre_indices.astype(jnp.int32)
    tidx = tags_indices.astype(jnp.int32)
    bf = jnp.bfloat16
    W1p = W1[640:]
    pw1 = price_W @ W1p
    b1c = (b1 + price_b @ W1p)[None, :]
    W1bf = W1.astype(bf)
    W2bf, W3bf, W4bf = W2.astype(bf), W3.astype(bf), W4.astype(bf)
    price2 = price[:, None]
    outs = []
    for s in range(S):
        v, g, t = _sc_gather(s, item, gidx, tidx,
                             item_table, genre_table, tags_table)
        outs.append(_tc_mlp(s, v, g, t, price2,
                            W1bf, pw1, b1c, W2bf, b2[None, :],
                            W3bf, b3[None, :], W4bf, b4[None, :]))
    return jnp.concatenate(outs, axis=0)
